# baseline (speedup 1.0000x reference)
import functools

import jax
import jax.numpy as jnp
from jax import lax
from jax.experimental import pallas as pl
from jax.experimental.pallas import tpu as pltpu
from jax.experimental.pallas import tpu_sc as plsc

_LANES = 16
_NUM_WORKERS = 32
_IDX_CHUNK = 128
_SEG = 128
_SUB = 32


def _ahat_body(chunk, num_users, uid_hbm, iid_hbm, ahat_hbm, av_hbm,
               uid_v, iid_v, row0_v, row1_v, avc_v, sem0, sem1):
    wid = lax.axis_index("s") * 2 + lax.axis_index("c")
    base = wid * chunk

    pltpu.sync_copy(uid_hbm.at[pl.ds(base, chunk)], uid_v)
    pltpu.sync_copy(iid_hbm.at[pl.ds(base, chunk)], iid_v)

    iota = lax.iota(jnp.int32, _LANES)
    rows = (row0_v, row1_v)
    sems = (sem0, sem1)
    nrounds = chunk // _SUB

    def fire_round(r, row_v, sem):
        r0 = r * _SUB

        def fire_group(gk, carry2):
            uvec = uid_v[pl.ds(r0 + gk * _LANES, _LANES)]
            ivec = iid_v[pl.ds(r0 + gk * _LANES, _LANES)]
            u0vec = (uvec // _SEG) * _SEG
            i0vec = (ivec // 8) * 8
            for k in range(_LANES):
                u0 = pl.multiple_of(u0vec[k], _SEG)
                i0 = pl.multiple_of(i0vec[k], 8)
                pltpu.async_copy(
                    ahat_hbm.at[pl.ds(i0, 8), pl.ds(u0, _SEG)],
                    row_v.at[pl.ds((gk * _LANES + k) * 8, 8)], sem)
            return carry2

        lax.fori_loop(0, _SUB // _LANES, fire_group, 0)

    def drain_extract(r, row_v, sem):
        r0 = r * _SUB
        pltpu.make_async_copy(ahat_hbm.at[pl.ds(0, _SUB * 8), pl.ds(0, _SEG)],
                              row_v, sem).wait()

        def extract(g, carry2):
            s = pl.ds(r0 + g * _LANES, _LANES)
            ridx = (iota + g * _LANES) * 8 + jnp.bitwise_and(iid_v[s], 7)
            lane = jnp.bitwise_and(uid_v[s], _SEG - 1)
            avc_v[s] = plsc.load_gather(row_v, [ridx, lane])
            return carry2

        lax.fori_loop(0, _SUB // _LANES, extract, 0, unroll=2)

    fire_round(0, rows[0], sems[0])

    def round_loop(r, carry):
        for p in range(2):
            @pl.when(lax.rem(r, 2) == p)
            def _():
                fire_round(r + 1, rows[1 - p], sems[1 - p])
                drain_extract(r, rows[p], sems[p])
        return carry

    lax.fori_loop(0, nrounds - 1, round_loop, 0)
    for p in range(2):
        @pl.when(lax.rem(nrounds - 1, 2) == p)
        def _():
            drain_extract(nrounds - 1, rows[p], sems[p])

    pltpu.sync_copy(avc_v, av_hbm.at[pl.ds(base, chunk)])


def _mf_body(chunk, embed,
             uid_hbm, iid_hbm, uemb_hbm, iemb_hbm, ubias_hbm, ibias_hbm,
             ucoe_hbm, out_hbm, coe_hbm,
             uid_v, iid_v, urows_v, irows_v, ub_v, ib_v, uc_v,
             out_v, sem):
    wid = lax.axis_index("s") * 2 + lax.axis_index("c")
    base = wid * chunk

    pltpu.sync_copy(uid_hbm.at[pl.ds(base, chunk)], uid_v)
    pltpu.sync_copy(iid_hbm.at[pl.ds(base, chunk)], iid_v)

    copies = [
        pltpu.async_copy(uemb_hbm.at[uid_v], urows_v, sem),
        pltpu.async_copy(iemb_hbm.at[iid_v], irows_v, sem),
        pltpu.async_copy(ubias_hbm.at[uid_v], ub_v, sem),
        pltpu.async_copy(ibias_hbm.at[iid_v], ib_v, sem),
        pltpu.async_copy(ucoe_hbm.at[uid_v], uc_v, sem),
    ]
    for c in copies:
        c.wait()

    iota = lax.iota(jnp.int32, _LANES)

    def group_loop(g, carry):
        b0 = g * _LANES
        s = pl.ds(b0, _LANES)
        bvec = iota + b0
        acc = ub_v[s] + ib_v[s]
        for e in range(embed):
            evec = jnp.full((_LANES,), e, jnp.int32)
            ue = plsc.load_gather(urows_v, [bvec, evec])
            ie = plsc.load_gather(irows_v, [bvec, evec])
            acc = acc + ue * ie
        out_v[s] = acc
        return carry

    lax.fori_loop(0, chunk // _LANES, group_loop, 0, unroll=2)

    pltpu.sync_copy(out_v, out_hbm.at[pl.ds(base, chunk)])
    pltpu.sync_copy(uc_v, coe_hbm.at[pl.ds(base, chunk)])


def kernel(uid, iid, user_emb, item_emb, user_bias, item_bias, user_coe,
           a_hat):
    batch = uid.shape[0]
    embed = user_emb.shape[1]
    chunk = batch // _NUM_WORKERS

    uid32 = uid.astype(jnp.int32)
    iid32 = iid.astype(jnp.int32)
    ucoe_flat = user_coe.reshape(-1)

    mesh = plsc.VectorSubcoreMesh(core_axis_name="c", subcore_axis_name="s")

    num_users = user_emb.shape[0]
    ahat_run = pl.kernel(
        functools.partial(_ahat_body, chunk, num_users),
        out_type=jax.ShapeDtypeStruct((batch,), jnp.float32),
        mesh=mesh,
        scratch_types=[
            pltpu.VMEM((chunk,), jnp.int32),
            pltpu.VMEM((chunk,), jnp.int32),
            pltpu.VMEM((_SUB * 8, _SEG), jnp.float32),
            pltpu.VMEM((_SUB * 8, _SEG), jnp.float32),
            pltpu.VMEM((chunk,), jnp.float32),
            pltpu.SemaphoreType.DMA,
            pltpu.SemaphoreType.DMA,
        ],
        compiler_params=pltpu.CompilerParams(
            needs_layout_passes=False, use_tc_tiling_on_sc=True,
            disable_bounds_checks=True),
    )
    av = ahat_run(uid32, iid32, a_hat.T)

    mf_run = pl.kernel(
        functools.partial(_mf_body, chunk, embed),
        out_type=(jax.ShapeDtypeStruct((batch,), jnp.float32),
                  jax.ShapeDtypeStruct((batch,), jnp.float32)),
        mesh=mesh,
        scratch_types=[
            pltpu.VMEM((chunk,), jnp.int32),
            pltpu.VMEM((chunk,), jnp.int32),
            pltpu.VMEM((chunk, embed), jnp.float32),
            pltpu.VMEM((chunk, embed), jnp.float32),
            pltpu.VMEM((chunk,), jnp.float32),
            pltpu.VMEM((chunk,), jnp.float32),
            pltpu.VMEM((chunk,), jnp.float32),
            pltpu.VMEM((chunk,), jnp.float32),
            pltpu.SemaphoreType.DMA,
        ],
        compiler_params=pltpu.CompilerParams(
            needs_layout_passes=False, use_tc_tiling_on_sc=False),
    )
    mf_out, coe = mf_run(uid32, iid32, user_emb, item_emb, user_bias,
                         item_bias, ucoe_flat)
    return mf_out + coe * av


# speedup vs baseline: 1.0034x; 1.0034x over previous
import functools

import jax
import jax.numpy as jnp
from jax import lax
from jax.experimental import pallas as pl
from jax.experimental.pallas import tpu as pltpu
from jax.experimental.pallas import tpu_sc as plsc

_LANES = 16
_NUM_WORKERS = 32
_IDX_CHUNK = 128
_SEG = 128
_SUB = 32


def _ahat_body(chunk, num_users, uid_hbm, iid_hbm, ahat_hbm, av_hbm,
               uid_v, iid_v, row0_v, row1_v, avc_v, sem0, sem1):
    wid = lax.axis_index("s") * 2 + lax.axis_index("c")
    base = wid * chunk

    pltpu.sync_copy(uid_hbm.at[pl.ds(base, chunk)], uid_v)
    pltpu.sync_copy(iid_hbm.at[pl.ds(base, chunk)], iid_v)

    iota = lax.iota(jnp.int32, _LANES)
    rows = (row0_v, row1_v)
    sems = (sem0, sem1)
    nrounds = chunk // _SUB

    def fire_round(r, row_v, sem):
        r0 = r * _SUB

        def fire_group(gk, carry2):
            uvec = uid_v[pl.ds(r0 + gk * _LANES, _LANES)]
            ivec = iid_v[pl.ds(r0 + gk * _LANES, _LANES)]
            u0vec = (uvec // _SEG) * _SEG
            i0vec = (ivec // 8) * 8
            for k in range(_LANES):
                u0 = pl.multiple_of(u0vec[k], _SEG)
                i0 = pl.multiple_of(i0vec[k], 8)
                pltpu.async_copy(
                    ahat_hbm.at[pl.ds(i0, 8), pl.ds(u0, _SEG)],
                    row_v.at[pl.ds((gk * _LANES + k) * 8, 8)], sem)
            return carry2

        lax.fori_loop(0, _SUB // _LANES, fire_group, 0)

    def drain_extract(r, row_v, sem):
        r0 = r * _SUB
        pltpu.make_async_copy(ahat_hbm.at[pl.ds(0, _SUB * 8), pl.ds(0, _SEG)],
                              row_v, sem).wait()

        def extract(g, carry2):
            s = pl.ds(r0 + g * _LANES, _LANES)
            ridx = (iota + g * _LANES) * 8 + jnp.bitwise_and(iid_v[s], 7)
            lane = jnp.bitwise_and(uid_v[s], _SEG - 1)
            avc_v[s] = plsc.load_gather(row_v, [ridx, lane])
            return carry2

        lax.fori_loop(0, _SUB // _LANES, extract, 0, unroll=2)

    fire_round(0, rows[0], sems[0])

    def round_loop(r, carry):
        for p in range(2):
            @pl.when(lax.rem(r, 2) == p)
            def _():
                fire_round(r + 1, rows[1 - p], sems[1 - p])
                drain_extract(r, rows[p], sems[p])
        return carry

    lax.fori_loop(0, nrounds - 1, round_loop, 0)
    for p in range(2):
        @pl.when(lax.rem(nrounds - 1, 2) == p)
        def _():
            drain_extract(nrounds - 1, rows[p], sems[p])

    pltpu.sync_copy(avc_v, av_hbm.at[pl.ds(base, chunk)])


def _mf_body(chunk, embed,
             uid_hbm, iid_hbm, uemb_hbm, iemb_hbm, ubias_hbm, ibias_hbm,
             ucoe_hbm, out_hbm, coe_hbm,
             uid_v, iid_v, urows_v, irows_v, ub_v, ib_v, uc_v,
             out_v, sem):
    wid = lax.axis_index("s") * 2 + lax.axis_index("c")
    base = wid * chunk

    pltpu.sync_copy(uid_hbm.at[pl.ds(base, chunk)], uid_v)
    pltpu.sync_copy(iid_hbm.at[pl.ds(base, chunk)], iid_v)

    copies = [
        pltpu.async_copy(uemb_hbm.at[uid_v], urows_v, sem),
        pltpu.async_copy(iemb_hbm.at[iid_v], irows_v, sem),
        pltpu.async_copy(ubias_hbm.at[uid_v], ub_v, sem),
        pltpu.async_copy(ibias_hbm.at[iid_v], ib_v, sem),
        pltpu.async_copy(ucoe_hbm.at[uid_v], uc_v, sem),
    ]
    for c in copies:
        c.wait()

    iota = lax.iota(jnp.int32, _LANES)

    def group_loop(g, carry):
        b0 = g * _LANES
        s = pl.ds(b0, _LANES)
        bvec = iota + b0
        zero = jnp.zeros((_LANES,), jnp.float32)
        parts = [ub_v[s], ib_v[s], zero, zero]
        for e in range(embed):
            evec = jnp.full((_LANES,), e, jnp.int32)
            ue = plsc.load_gather(urows_v, [bvec, evec])
            ie = plsc.load_gather(irows_v, [bvec, evec])
            parts[e % 4] = parts[e % 4] + ue * ie
        out_v[s] = (parts[0] + parts[1]) + (parts[2] + parts[3])
        return carry

    lax.fori_loop(0, chunk // _LANES, group_loop, 0, unroll=2)

    pltpu.sync_copy(out_v, out_hbm.at[pl.ds(base, chunk)])
    pltpu.sync_copy(uc_v, coe_hbm.at[pl.ds(base, chunk)])


def kernel(uid, iid, user_emb, item_emb, user_bias, item_bias, user_coe,
           a_hat):
    batch = uid.shape[0]
    embed = user_emb.shape[1]
    chunk = batch // _NUM_WORKERS

    uid32 = uid.astype(jnp.int32)
    iid32 = iid.astype(jnp.int32)
    ucoe_flat = user_coe.reshape(-1)

    mesh = plsc.VectorSubcoreMesh(core_axis_name="c", subcore_axis_name="s")

    num_users = user_emb.shape[0]
    ahat_run = pl.kernel(
        functools.partial(_ahat_body, chunk, num_users),
        out_type=jax.ShapeDtypeStruct((batch,), jnp.float32),
        mesh=mesh,
        scratch_types=[
            pltpu.VMEM((chunk,), jnp.int32),
            pltpu.VMEM((chunk,), jnp.int32),
            pltpu.VMEM((_SUB * 8, _SEG), jnp.float32),
            pltpu.VMEM((_SUB * 8, _SEG), jnp.float32),
            pltpu.VMEM((chunk,), jnp.float32),
            pltpu.SemaphoreType.DMA,
            pltpu.SemaphoreType.DMA,
        ],
        compiler_params=pltpu.CompilerParams(
            needs_layout_passes=False, use_tc_tiling_on_sc=True,
            disable_bounds_checks=True),
    )
    av = ahat_run(uid32, iid32, a_hat.T)

    mf_run = pl.kernel(
        functools.partial(_mf_body, chunk, embed),
        out_type=(jax.ShapeDtypeStruct((batch,), jnp.float32),
                  jax.ShapeDtypeStruct((batch,), jnp.float32)),
        mesh=mesh,
        scratch_types=[
            pltpu.VMEM((chunk,), jnp.int32),
            pltpu.VMEM((chunk,), jnp.int32),
            pltpu.VMEM((chunk, embed), jnp.float32),
            pltpu.VMEM((chunk, embed), jnp.float32),
            pltpu.VMEM((chunk,), jnp.float32),
            pltpu.VMEM((chunk,), jnp.float32),
            pltpu.VMEM((chunk,), jnp.float32),
            pltpu.VMEM((chunk,), jnp.float32),
            pltpu.SemaphoreType.DMA,
        ],
        compiler_params=pltpu.CompilerParams(
            needs_layout_passes=False, use_tc_tiling_on_sc=False,
            disable_bounds_checks=True),
    )
    mf_out, coe = mf_run(uid32, iid32, user_emb, item_emb, user_bias,
                         item_bias, ucoe_flat)
    return mf_out + coe * av


# speedup vs baseline: 1.1420x; 1.1381x over previous
import functools

import jax
import jax.numpy as jnp
from jax import lax
from jax.experimental import pallas as pl
from jax.experimental.pallas import tpu as pltpu
from jax.experimental.pallas import tpu_sc as plsc

_LANES = 16
_NUM_WORKERS = 32
_IDX_CHUNK = 128
_SEG = 128
_SUB = 32


def _ahat_body(chunk, num_users, uid_hbm, iid_hbm, ahat_hbm, av_hbm,
               uid_v, iid_v, row0_v, row1_v, avc_v, sem0, sem1):
    wid = lax.axis_index("s") * 2 + lax.axis_index("c")
    base = wid * chunk

    pltpu.sync_copy(uid_hbm.at[pl.ds(base, chunk)], uid_v)
    pltpu.sync_copy(iid_hbm.at[pl.ds(base, chunk)], iid_v)

    iota = lax.iota(jnp.int32, _LANES)
    rows = (row0_v, row1_v)
    sems = (sem0, sem1)
    nrounds = chunk // _SUB

    def fire_round(r, row_v, sem):
        r0 = r * _SUB

        def fire_group(gk, carry2):
            uvec = uid_v[pl.ds(r0 + gk * _LANES, _LANES)]
            ivec = iid_v[pl.ds(r0 + gk * _LANES, _LANES)]
            u0vec = (uvec // _SEG) * _SEG
            i0vec = (ivec // 8) * 8
            for k in range(_LANES):
                u0 = pl.multiple_of(u0vec[k], _SEG)
                i0 = pl.multiple_of(i0vec[k], 8)
                pltpu.async_copy(
                    ahat_hbm.at[pl.ds(i0, 8), pl.ds(u0, _SEG)],
                    row_v.at[pl.ds((gk * _LANES + k) * 8, 8)], sem)
            return carry2

        lax.fori_loop(0, _SUB // _LANES, fire_group, 0)

    def drain_extract(r, row_v, sem):
        r0 = r * _SUB
        pltpu.make_async_copy(ahat_hbm.at[pl.ds(0, _SUB * 8), pl.ds(0, _SEG)],
                              row_v, sem).wait()

        def extract(g, carry2):
            s = pl.ds(r0 + g * _LANES, _LANES)
            ridx = (iota + g * _LANES) * 8 + jnp.bitwise_and(iid_v[s], 7)
            lane = jnp.bitwise_and(uid_v[s], _SEG - 1)
            avc_v[s] = plsc.load_gather(row_v, [ridx, lane])
            return carry2

        lax.fori_loop(0, _SUB // _LANES, extract, 0, unroll=2)

    fire_round(0, rows[0], sems[0])

    def round_loop(r, carry):
        for p in range(2):
            @pl.when(lax.rem(r, 2) == p)
            def _():
                fire_round(r + 1, rows[1 - p], sems[1 - p])
                drain_extract(r, rows[p], sems[p])
        return carry

    lax.fori_loop(0, nrounds - 1, round_loop, 0)
    for p in range(2):
        @pl.when(lax.rem(nrounds - 1, 2) == p)
        def _():
            drain_extract(nrounds - 1, rows[p], sems[p])

    pltpu.sync_copy(avc_v, av_hbm.at[pl.ds(base, chunk)])


def _mf_body(chunk, embed,
             uid_hbm, iid_hbm, uemb_hbm, iemb_hbm, ubias_hbm, ibias_hbm,
             ucoe_hbm, out_hbm, coe_hbm,
             uid_v, iid_v, urows_v, irows_v, ub_v, ib_v, uc_v,
             out_v, sem):
    wid = lax.axis_index("s") * 2 + lax.axis_index("c")
    base = wid * chunk

    pltpu.sync_copy(uid_hbm.at[pl.ds(base, chunk)], uid_v)
    pltpu.sync_copy(iid_hbm.at[pl.ds(base, chunk)], iid_v)

    copies = [
        pltpu.async_copy(uemb_hbm.at[uid_v], urows_v, sem),
        pltpu.async_copy(iemb_hbm.at[iid_v], irows_v, sem),
        pltpu.async_copy(ubias_hbm.at[uid_v], ub_v, sem),
        pltpu.async_copy(ibias_hbm.at[iid_v], ib_v, sem),
        pltpu.async_copy(ucoe_hbm.at[uid_v], uc_v, sem),
    ]
    for c in copies:
        c.wait()

    iota = lax.iota(jnp.int32, _LANES)

    def group_loop(g, carry):
        b0 = g * _LANES
        s = pl.ds(b0, _LANES)
        bvec = iota + b0
        zero = jnp.zeros((_LANES,), jnp.float32)
        parts = [ub_v[s], ib_v[s], zero, zero]
        for e in range(embed):
            evec = jnp.bitwise_and(iota + e, embed - 1)
            ue = plsc.load_gather(urows_v, [bvec, evec])
            ie = plsc.load_gather(irows_v, [bvec, evec])
            parts[e % 4] = parts[e % 4] + ue * ie
        out_v[s] = (parts[0] + parts[1]) + (parts[2] + parts[3])
        return carry

    lax.fori_loop(0, chunk // _LANES, group_loop, 0, unroll=2)

    pltpu.sync_copy(out_v, out_hbm.at[pl.ds(base, chunk)])
    pltpu.sync_copy(uc_v, coe_hbm.at[pl.ds(base, chunk)])


def kernel(uid, iid, user_emb, item_emb, user_bias, item_bias, user_coe,
           a_hat):
    batch = uid.shape[0]
    embed = user_emb.shape[1]
    chunk = batch // _NUM_WORKERS

    uid32 = uid.astype(jnp.int32)
    iid32 = iid.astype(jnp.int32)
    ucoe_flat = user_coe.reshape(-1)

    mesh = plsc.VectorSubcoreMesh(core_axis_name="c", subcore_axis_name="s")

    num_users = user_emb.shape[0]
    ahat_run = pl.kernel(
        functools.partial(_ahat_body, chunk, num_users),
        out_type=jax.ShapeDtypeStruct((batch,), jnp.float32),
        mesh=mesh,
        scratch_types=[
            pltpu.VMEM((chunk,), jnp.int32),
            pltpu.VMEM((chunk,), jnp.int32),
            pltpu.VMEM((_SUB * 8, _SEG), jnp.float32),
            pltpu.VMEM((_SUB * 8, _SEG), jnp.float32),
            pltpu.VMEM((chunk,), jnp.float32),
            pltpu.SemaphoreType.DMA,
            pltpu.SemaphoreType.DMA,
        ],
        compiler_params=pltpu.CompilerParams(
            needs_layout_passes=False, use_tc_tiling_on_sc=True,
            disable_bounds_checks=True),
    )
    av = ahat_run(uid32, iid32, a_hat.T)

    mf_run = pl.kernel(
        functools.partial(_mf_body, chunk, embed),
        out_type=(jax.ShapeDtypeStruct((batch,), jnp.float32),
                  jax.ShapeDtypeStruct((batch,), jnp.float32)),
        mesh=mesh,
        scratch_types=[
            pltpu.VMEM((chunk,), jnp.int32),
            pltpu.VMEM((chunk,), jnp.int32),
            pltpu.VMEM((chunk, embed), jnp.float32),
            pltpu.VMEM((chunk, embed), jnp.float32),
            pltpu.VMEM((chunk,), jnp.float32),
            pltpu.VMEM((chunk,), jnp.float32),
            pltpu.VMEM((chunk,), jnp.float32),
            pltpu.VMEM((chunk,), jnp.float32),
            pltpu.SemaphoreType.DMA,
        ],
        compiler_params=pltpu.CompilerParams(
            needs_layout_passes=False, use_tc_tiling_on_sc=False,
            disable_bounds_checks=True),
    )
    mf_out, coe = mf_run(uid32, iid32, user_emb, item_emb, user_bias,
                         item_bias, ucoe_flat)
    return mf_out + coe * av


# speedup vs baseline: 1.1427x; 1.0006x over previous
import functools

import jax
import jax.numpy as jnp
from jax import lax
from jax.experimental import pallas as pl
from jax.experimental.pallas import tpu as pltpu
from jax.experimental.pallas import tpu_sc as plsc

_LANES = 16
_NUM_WORKERS = 32
_IDX_CHUNK = 128
_SEG = 128
_SUB = 32


def _ahat_body(chunk, num_users, uid_hbm, iid_hbm, ahat_hbm, av_hbm,
               uid_v, iid_v, row0_v, row1_v, avc_v, sem0, sem1):
    wid = lax.axis_index("s") * 2 + lax.axis_index("c")
    base = wid * chunk

    pltpu.sync_copy(uid_hbm.at[pl.ds(base, chunk)], uid_v)
    pltpu.sync_copy(iid_hbm.at[pl.ds(base, chunk)], iid_v)

    iota = lax.iota(jnp.int32, _LANES)
    rows = (row0_v, row1_v)
    sems = (sem0, sem1)
    nrounds = chunk // _SUB

    def fire_round(r, row_v, sem):
        r0 = r * _SUB

        def fire_group(gk, carry2):
            uvec = uid_v[pl.ds(r0 + gk * _LANES, _LANES)]
            ivec = iid_v[pl.ds(r0 + gk * _LANES, _LANES)]
            u0vec = (uvec // _SEG) * _SEG
            i0vec = (ivec // 8) * 8
            for k in range(_LANES):
                u0 = pl.multiple_of(u0vec[k], _SEG)
                i0 = pl.multiple_of(i0vec[k], 8)
                pltpu.async_copy(
                    ahat_hbm.at[pl.ds(i0, 8), pl.ds(u0, _SEG)],
                    row_v.at[pl.ds((gk * _LANES + k) * 8, 8)], sem)
            return carry2

        lax.fori_loop(0, _SUB // _LANES, fire_group, 0)

    def drain_extract(r, row_v, sem):
        r0 = r * _SUB
        pltpu.make_async_copy(ahat_hbm.at[pl.ds(0, _SUB * 8), pl.ds(0, _SEG)],
                              row_v, sem).wait()

        def extract(g, carry2):
            s = pl.ds(r0 + g * _LANES, _LANES)
            ridx = (iota + g * _LANES) * 8 + jnp.bitwise_and(iid_v[s], 7)
            lane = jnp.bitwise_and(uid_v[s], _SEG - 1)
            avc_v[s] = plsc.load_gather(row_v, [ridx, lane])
            return carry2

        lax.fori_loop(0, _SUB // _LANES, extract, 0, unroll=2)

    fire_round(0, rows[0], sems[0])

    def round_loop(r, carry):
        for p in range(2):
            @pl.when(lax.rem(r, 2) == p)
            def _():
                fire_round(r + 1, rows[1 - p], sems[1 - p])
                drain_extract(r, rows[p], sems[p])
        return carry

    lax.fori_loop(0, nrounds - 1, round_loop, 0)
    for p in range(2):
        @pl.when(lax.rem(nrounds - 1, 2) == p)
        def _():
            drain_extract(nrounds - 1, rows[p], sems[p])

    pltpu.sync_copy(avc_v, av_hbm.at[pl.ds(base, chunk)])


def _mf_body(chunk, embed,
             uid_hbm, iid_hbm, uemb_hbm, iemb_hbm, ubias_hbm, ibias_hbm,
             ucoe_hbm, out_hbm, coe_hbm,
             uid_v, iid_v, urows_v, irows_v, ub_v, ib_v, uc_v,
             out_v, sem):
    wid = lax.axis_index("s") * 2 + lax.axis_index("c")
    base = wid * chunk

    pltpu.sync_copy(uid_hbm.at[pl.ds(base, chunk)], uid_v)
    pltpu.sync_copy(iid_hbm.at[pl.ds(base, chunk)], iid_v)

    copies = [
        pltpu.async_copy(uemb_hbm.at[uid_v], urows_v, sem),
        pltpu.async_copy(iemb_hbm.at[iid_v], irows_v, sem),
        pltpu.async_copy(ubias_hbm.at[uid_v], ub_v, sem),
        pltpu.async_copy(ibias_hbm.at[iid_v], ib_v, sem),
        pltpu.async_copy(ucoe_hbm.at[uid_v], uc_v, sem),
    ]
    for c in copies:
        c.wait()

    iota = lax.iota(jnp.int32, _LANES)

    def group_loop(g, carry):
        b0 = g * _LANES
        s = pl.ds(b0, _LANES)
        bvec = iota + b0
        zero = jnp.zeros((_LANES,), jnp.float32)
        parts = [ub_v[s], ib_v[s], zero, zero]
        for e in range(embed):
            evec = jnp.bitwise_and(iota + e, embed - 1)
            ue = plsc.load_gather(urows_v, [bvec, evec])
            ie = plsc.load_gather(irows_v, [bvec, evec])
            parts[e % 4] = parts[e % 4] + ue * ie
        out_v[s] = (parts[0] + parts[1]) + (parts[2] + parts[3])
        return carry

    lax.fori_loop(0, chunk // _LANES, group_loop, 0, unroll=2)

    pltpu.sync_copy(out_v, out_hbm.at[pl.ds(base, chunk)])
    pltpu.sync_copy(uc_v, coe_hbm.at[pl.ds(base, chunk)])


def kernel(uid, iid, user_emb, item_emb, user_bias, item_bias, user_coe,
           a_hat):
    batch = uid.shape[0]
    embed = user_emb.shape[1]
    chunk = batch // _NUM_WORKERS

    uid32 = uid.astype(jnp.int32)
    iid32 = iid.astype(jnp.int32)
    ucoe_flat = user_coe.reshape(-1)
    uemb_lin = lax.optimization_barrier(user_emb.reshape(-1)).reshape(
        user_emb.shape)
    iemb_lin = lax.optimization_barrier(item_emb.reshape(-1)).reshape(
        item_emb.shape)

    mesh = plsc.VectorSubcoreMesh(core_axis_name="c", subcore_axis_name="s")

    num_users = user_emb.shape[0]
    ahat_run = pl.kernel(
        functools.partial(_ahat_body, chunk, num_users),
        out_type=jax.ShapeDtypeStruct((batch,), jnp.float32),
        mesh=mesh,
        scratch_types=[
            pltpu.VMEM((chunk,), jnp.int32),
            pltpu.VMEM((chunk,), jnp.int32),
            pltpu.VMEM((_SUB * 8, _SEG), jnp.float32),
            pltpu.VMEM((_SUB * 8, _SEG), jnp.float32),
            pltpu.VMEM((chunk,), jnp.float32),
            pltpu.SemaphoreType.DMA,
            pltpu.SemaphoreType.DMA,
        ],
        compiler_params=pltpu.CompilerParams(
            needs_layout_passes=False, use_tc_tiling_on_sc=True,
            disable_bounds_checks=True),
    )
    av = ahat_run(uid32, iid32, a_hat.T)

    mf_run = pl.kernel(
        functools.partial(_mf_body, chunk, embed),
        out_type=(jax.ShapeDtypeStruct((batch,), jnp.float32),
                  jax.ShapeDtypeStruct((batch,), jnp.float32)),
        mesh=mesh,
        scratch_types=[
            pltpu.VMEM((chunk,), jnp.int32),
            pltpu.VMEM((chunk,), jnp.int32),
            pltpu.VMEM((chunk, embed), jnp.float32),
            pltpu.VMEM((chunk, embed), jnp.float32),
            pltpu.VMEM((chunk,), jnp.float32),
            pltpu.VMEM((chunk,), jnp.float32),
            pltpu.VMEM((chunk,), jnp.float32),
            pltpu.VMEM((chunk,), jnp.float32),
            pltpu.SemaphoreType.DMA,
        ],
        compiler_params=pltpu.CompilerParams(
            needs_layout_passes=False, use_tc_tiling_on_sc=False,
            disable_bounds_checks=True),
    )
    mf_out, coe = mf_run(uid32, iid32, uemb_lin, iemb_lin, user_bias,
                         item_bias, ucoe_flat)
    return mf_out + coe * av


# speedup vs baseline: 1.1557x; 1.0114x over previous
import functools

import jax
import jax.numpy as jnp
from jax import lax
from jax.experimental import pallas as pl
from jax.experimental.pallas import tpu as pltpu
from jax.experimental.pallas import tpu_sc as plsc

_LANES = 16
_NUM_WORKERS = 32
_IDX_CHUNK = 128
_SEG = 128
_SUB = 32


def _ahat_body(chunk, num_users, uid_hbm, iid_hbm, ahat_hbm, av_hbm,
               uid_v, iid_v, row0_v, row1_v, avc_v, sem0, sem1):
    wid = lax.axis_index("s") * 2 + lax.axis_index("c")
    base = wid * chunk

    pltpu.sync_copy(uid_hbm.at[pl.ds(base, chunk)], uid_v)
    pltpu.sync_copy(iid_hbm.at[pl.ds(base, chunk)], iid_v)

    iota = lax.iota(jnp.int32, _LANES)
    rows = (row0_v, row1_v)
    sems = (sem0, sem1)
    nrounds = chunk // _SUB

    def fire_round(r, row_v, sem):
        r0 = r * _SUB

        def fire_group(gk, carry2):
            uvec = uid_v[pl.ds(r0 + gk * _LANES, _LANES)]
            ivec = iid_v[pl.ds(r0 + gk * _LANES, _LANES)]
            u0vec = (uvec // _SEG) * _SEG
            i0vec = (ivec // 8) * 8
            for k in range(_LANES):
                u0 = pl.multiple_of(u0vec[k], _SEG)
                i0 = pl.multiple_of(i0vec[k], 8)
                pltpu.async_copy(
                    ahat_hbm.at[pl.ds(i0, 8), pl.ds(u0, _SEG)],
                    row_v.at[pl.ds((gk * _LANES + k) * 8, 8)], sem)
            return carry2

        lax.fori_loop(0, _SUB // _LANES, fire_group, 0)

    def drain_extract(r, row_v, sem):
        r0 = r * _SUB
        pltpu.make_async_copy(ahat_hbm.at[pl.ds(0, _SUB * 8), pl.ds(0, _SEG)],
                              row_v, sem).wait()

        def extract(g, carry2):
            s = pl.ds(r0 + g * _LANES, _LANES)
            ridx = (iota + g * _LANES) * 8 + jnp.bitwise_and(iid_v[s], 7)
            lane = jnp.bitwise_and(uid_v[s], _SEG - 1)
            avc_v[s] = plsc.load_gather(row_v, [ridx, lane])
            return carry2

        lax.fori_loop(0, _SUB // _LANES, extract, 0, unroll=2)

    fire_round(0, rows[0], sems[0])

    def round_loop(r, carry):
        for p in range(2):
            @pl.when(lax.rem(r, 2) == p)
            def _():
                fire_round(r + 1, rows[1 - p], sems[1 - p])
                drain_extract(r, rows[p], sems[p])
        return carry

    lax.fori_loop(0, nrounds - 1, round_loop, 0)
    for p in range(2):
        @pl.when(lax.rem(nrounds - 1, 2) == p)
        def _():
            drain_extract(nrounds - 1, rows[p], sems[p])

    pltpu.sync_copy(avc_v, av_hbm.at[pl.ds(base, chunk)])


def _mf_body(chunk, embed,
             uid_hbm, iid_hbm, uemb_hbm, iemb_hbm, ubias_hbm, ibias_hbm,
             ucoe_hbm, out_hbm, coe_hbm,
             uid_v, iid_v, urows_v, irows_v, ub_v, ib_v, uc_v,
             out_v, sem):
    wid = lax.axis_index("s") * 2 + lax.axis_index("c")
    base = wid * chunk

    pltpu.sync_copy(uid_hbm.at[pl.ds(base, chunk)], uid_v)
    pltpu.sync_copy(iid_hbm.at[pl.ds(base, chunk)], iid_v)

    copies = [
        pltpu.async_copy(uemb_hbm.at[uid_v], urows_v, sem),
        pltpu.async_copy(iemb_hbm.at[iid_v], irows_v, sem),
        pltpu.async_copy(ubias_hbm.at[uid_v], ub_v, sem),
        pltpu.async_copy(ibias_hbm.at[iid_v], ib_v, sem),
        pltpu.async_copy(ucoe_hbm.at[uid_v], uc_v, sem),
    ]
    for c in copies:
        c.wait()

    iota = lax.iota(jnp.int32, _LANES)

    def group_loop(g, carry):
        b0 = g * _LANES
        s = pl.ds(b0, _LANES)
        bvec = iota + b0
        zero = jnp.zeros((_LANES,), jnp.float32)
        parts = [ub_v[s], ib_v[s], zero, zero]
        for e in range(embed):
            evec = jnp.bitwise_and(iota + e, embed - 1)
            ue = plsc.load_gather(urows_v, [bvec, evec])
            ie = plsc.load_gather(irows_v, [bvec, evec])
            parts[e % 4] = parts[e % 4] + ue * ie
        out_v[s] = (parts[0] + parts[1]) + (parts[2] + parts[3])
        return carry

    lax.fori_loop(0, chunk // _LANES, group_loop, 0, unroll=2)

    pltpu.sync_copy(out_v, out_hbm.at[pl.ds(base, chunk)])
    pltpu.sync_copy(uc_v, coe_hbm.at[pl.ds(base, chunk)])


def kernel(uid, iid, user_emb, item_emb, user_bias, item_bias, user_coe,
           a_hat):
    batch = uid.shape[0]
    embed = user_emb.shape[1]
    chunk = batch // _NUM_WORKERS

    uid32 = uid.astype(jnp.int32)
    iid32 = iid.astype(jnp.int32)
    one = (iid32[0] * 0 + 1).astype(jnp.float32)
    uemb_lin = lax.optimization_barrier(user_emb.reshape(-1)).reshape(
        user_emb.shape)
    iemb_lin = (item_emb.reshape(-1) * one).reshape(item_emb.shape)
    ucoe_flat = user_coe.reshape(-1) * one

    mesh = plsc.VectorSubcoreMesh(core_axis_name="c", subcore_axis_name="s")

    num_users = user_emb.shape[0]
    ahat_run = pl.kernel(
        functools.partial(_ahat_body, chunk, num_users),
        out_type=jax.ShapeDtypeStruct((batch,), jnp.float32),
        mesh=mesh,
        scratch_types=[
            pltpu.VMEM((chunk,), jnp.int32),
            pltpu.VMEM((chunk,), jnp.int32),
            pltpu.VMEM((_SUB * 8, _SEG), jnp.float32),
            pltpu.VMEM((_SUB * 8, _SEG), jnp.float32),
            pltpu.VMEM((chunk,), jnp.float32),
            pltpu.SemaphoreType.DMA,
            pltpu.SemaphoreType.DMA,
        ],
        compiler_params=pltpu.CompilerParams(
            needs_layout_passes=False, use_tc_tiling_on_sc=True,
            disable_bounds_checks=True),
    )
    av = ahat_run(uid32, iid32, a_hat.T)

    mf_run = pl.kernel(
        functools.partial(_mf_body, chunk, embed),
        out_type=(jax.ShapeDtypeStruct((batch,), jnp.float32),
                  jax.ShapeDtypeStruct((batch,), jnp.float32)),
        mesh=mesh,
        scratch_types=[
            pltpu.VMEM((chunk,), jnp.int32),
            pltpu.VMEM((chunk,), jnp.int32),
            pltpu.VMEM((chunk, embed), jnp.float32),
            pltpu.VMEM((chunk, embed), jnp.float32),
            pltpu.VMEM((chunk,), jnp.float32),
            pltpu.VMEM((chunk,), jnp.float32),
            pltpu.VMEM((chunk,), jnp.float32),
            pltpu.VMEM((chunk,), jnp.float32),
            pltpu.SemaphoreType.DMA,
        ],
        compiler_params=pltpu.CompilerParams(
            needs_layout_passes=False, use_tc_tiling_on_sc=False,
            disable_bounds_checks=True),
    )
    mf_out, coe = mf_run(uid32, iid32, uemb_lin, iemb_lin, user_bias,
                         item_bias, ucoe_flat)
    return mf_out + coe * av
